# Initial kernel scaffold; baseline (speedup 1.0000x reference)
import jax
import jax.numpy as jnp
from jax.experimental import pallas as pl


def kernel(x, edge_index, w_gcn_0, w_gcn_1, w_rlu):
    raise NotImplementedError("write your pallas kernel here")


# speedup vs baseline: 30.5719x; 30.5719x over previous
import functools

import jax
import jax.numpy as jnp
from jax import lax
from jax.experimental import pallas as pl
from jax.experimental.pallas import tpu as pltpu
from jax.experimental.pallas import tpu_sc as plsc

M = 10000
E = 320000
FIN = 128
B = 8
K = 3

NT = 16
SR = 640
M_P = SR * NT
PTE = E // NT
CH = 800
NCH = PTE // CH

_MAGIC = 0x5F3759DF

_mesh = plsc.VectorSubcoreMesh(
    core_axis_name="c", subcore_axis_name="s", num_cores=1, num_subcores=NT
)

_sc_scratch = [
    pltpu.VMEM_SHARED((M_P,), jnp.float32),
    pltpu.VMEM_SHARED((M_P,), jnp.float32),
    pltpu.VMEM_SHARED((M_P, 16), jnp.float32),
    pltpu.VMEM_SHARED((M_P, 16), jnp.float32),
    pltpu.VMEM_SHARED((M_P, 16), jnp.float32),
    pltpu.VMEM_SHARED((M_P, 16), jnp.float32),
    pltpu.VMEM((CH,), jnp.int32),
    pltpu.VMEM((CH,), jnp.int32),
    pltpu.VMEM((CH, 16), jnp.float32),
    pltpu.VMEM((CH,), jnp.float32),
    pltpu.VMEM((SR,), jnp.float32),
    pltpu.VMEM((SR, 16), jnp.float32),
    pltpu.VMEM((SR, 16), jnp.float32),
    pltpu.VMEM((SR, 16), jnp.float32),
    pltpu.VMEM((SR, 16), jnp.float32),
    pltpu.VMEM((SR, 16), jnp.float32),
    pltpu.VMEM((SR, 16), jnp.float32),
    pltpu.VMEM((SR,), jnp.float32),
    pltpu.VMEM((SR,), jnp.float32),
    pltpu.VMEM((4, 16), jnp.float32),
]


@functools.partial(
    pl.kernel,
    out_type=jax.ShapeDtypeStruct((M_P, 16), jnp.float32),
    mesh=_mesh,
    scratch_types=_sc_scratch,
    compiler_params=pltpu.CompilerParams(
        needs_layout_passes=False, use_tc_tiling_on_sc=False
    ),
)
def _sc_cheb(rows_hbm, cols_hbm, za_hbm, zb_hbm, cons_hbm, out_hbm,
             sp_deg_r, sp_deg_c, sp_in, sp_acc1, sp_acc2, sp_h,
             ridx, cidx, gbuf, ones_v, zb1, zeros_v,
             wa, wb, wc, invr16, invc16, wdr, wdc, cons_v):
    t = lax.axis_index("s")
    ebase = t * PTE
    rbase = t * SR
    i16 = lax.iota(jnp.int32, 16)
    perm8 = jnp.bitwise_and(i16 + 8, 15)

    def rot8(v):
        return jnp.take_along_axis(
            v, perm8, axis=0, mode=lax.GatherScatterMode.PROMISE_IN_BOUNDS
        )

    def fill1(ref, n, val):
        def bd(i, _):
            ref[pl.ds(i * 16, 16)] = jnp.full((16,), val, jnp.float32)
            return 0
        lax.fori_loop(0, n, bd, 0)

    def fill2(ref, val):
        def bd(i, _):
            ref[i] = jnp.full((16,), val, jnp.float32)
            return 0
        lax.fori_loop(0, SR, bd, 0)

    def stripe(sp2d):
        return sp2d.at[pl.ds(rbase, SR)]

    def zero_acc(sp2d):
        pltpu.sync_copy(zeros_v, stripe(sp2d))

    def ew(body):
        def bd(i, _):
            body(i)
            return 0
        lax.fori_loop(0, SR, bd, 0)

    def s_pass(acc2d):
        def ch_body(ci, _):
            off = ebase + ci * CH
            pltpu.sync_copy(cols_hbm.at[pl.ds(off, CH)], cidx)
            pltpu.sync_copy(sp_in.at[cidx], gbuf)
            pltpu.sync_copy(rows_hbm.at[pl.ds(off, CH)], ridx)
            pltpu.sync_copy(gbuf, acc2d.at[ridx], add=True)
            return 0
        lax.fori_loop(0, NCH, ch_body, 0)

    fill1(ones_v, CH // 16, 1.0)
    fill1(zb1, SR // 16, 0.0)
    fill2(zeros_v, 0.0)
    pltpu.sync_copy(cons_hbm, cons_v)
    pltpu.sync_copy(zb1, sp_deg_r.at[pl.ds(rbase, SR)])
    pltpu.sync_copy(zb1, sp_deg_c.at[pl.ds(rbase, SR)])
    plsc.subcore_barrier()

    def deg_chunk(ci, _):
        off = ebase + ci * CH
        pltpu.sync_copy(rows_hbm.at[pl.ds(off, CH)], ridx)
        pltpu.sync_copy(cols_hbm.at[pl.ds(off, CH)], cidx)
        pltpu.sync_copy(ones_v, sp_deg_r.at[ridx], add=True)
        pltpu.sync_copy(ones_v, sp_deg_c.at[cidx], add=True)
        return 0
    lax.fori_loop(0, NCH, deg_chunk, 0)
    plsc.subcore_barrier()

    pltpu.sync_copy(sp_deg_r.at[pl.ds(rbase, SR)], wdr)
    pltpu.sync_copy(sp_deg_c.at[pl.ds(rbase, SR)], wdc)

    def inv_ref(ref):
        def bd(i, _):
            s = pl.ds(i * 16, 16)
            x = jnp.maximum(ref[s], 1.0)
            yi = (jnp.full((16,), _MAGIC, jnp.int32)
                  - lax.shift_right_logical(plsc.bitcast(x, jnp.int32), 1))
            y = plsc.bitcast(yi, jnp.float32)
            y = y * (1.5 - 0.5 * x * y * y)
            y = y * (1.5 - 0.5 * x * y * y)
            y = y * (1.5 - 0.5 * x * y * y)
            ref[s] = y
            return 0
        lax.fori_loop(0, SR // 16, bd, 0)
    inv_ref(wdr)
    inv_ref(wdc)

    def expand(i, _):
        sp = jnp.full((16,), i, jnp.int32)
        vr = plsc.load_gather(wdr, [sp])
        vc = plsc.load_gather(wdc, [sp])
        invr16[i] = vr
        invc16[i] = vc
        return 0
    lax.fori_loop(0, SR, expand, 0)

    rr = cons_v[0]
    c1 = cons_v[1]
    c2 = cons_v[2]
    c3 = cons_v[3]

    pltpu.sync_copy(za_hbm.at[pl.ds(rbase, SR)], wa)
    ew(lambda i: wa.__setitem__(i, wa[i] * invc16[i]))
    pltpu.sync_copy(wa, stripe(sp_in))
    zero_acc(sp_acc1)
    plsc.subcore_barrier()
    s_pass(sp_acc1)
    plsc.subcore_barrier()

    pltpu.sync_copy(stripe(sp_acc1), wa)
    ew(lambda i: wb.__setitem__(i, invr16[i] * invc16[i] * rot8(wa[i])))
    pltpu.sync_copy(wb, stripe(sp_in))
    zero_acc(sp_acc2)
    plsc.subcore_barrier()
    s_pass(sp_acc2)
    plsc.subcore_barrier()

    pltpu.sync_copy(zb_hbm.at[pl.ds(rbase, SR)], wa)
    pltpu.sync_copy(stripe(sp_acc1), wb)
    pltpu.sync_copy(stripe(sp_acc2), wc)
    ew(lambda i: wa.__setitem__(
        i, jnp.maximum(
            wa[i] - invr16[i] * wb[i] + 2.0 * invr16[i] * wc[i] + rr, 0.0)))
    pltpu.sync_copy(wa, stripe(sp_h))
    ew(lambda i: wb.__setitem__(i, invc16[i] * wa[i]))
    pltpu.sync_copy(wb, stripe(sp_in))
    zero_acc(sp_acc1)
    plsc.subcore_barrier()
    s_pass(sp_acc1)
    plsc.subcore_barrier()

    pltpu.sync_copy(stripe(sp_acc1), wb)
    ew(lambda i: wb.__setitem__(i, invr16[i] * invc16[i] * wb[i]))
    pltpu.sync_copy(wb, stripe(sp_in))
    zero_acc(sp_acc2)
    plsc.subcore_barrier()
    s_pass(sp_acc2)
    plsc.subcore_barrier()

    pltpu.sync_copy(stripe(sp_h), wa)
    pltpu.sync_copy(stripe(sp_acc1), wb)
    pltpu.sync_copy(stripe(sp_acc2), wc)
    ew(lambda i: wa.__setitem__(
        i, jnp.maximum(
            c1 * wa[i] + invr16[i] * (c3 * wc[i] - c2 * wb[i]) + rr, 0.0)))
    pltpu.sync_copy(wa, out_hbm.at[pl.ds(rbase, SR)])


def _proj_body(x_ref, w_ref, o_ref):
    pm = lax.dot_general(
        x_ref[0], w_ref[...], (((1,), (0,)), ((), ())),
        preferred_element_type=jnp.float32,
    )
    q = jnp.concatenate(
        [pm[:, 1:2], pm[:, 2:3], pm[:, 0:1] - pm[:, 2:3]], axis=1
    )
    o_ref[...] = q[None]


_BM = 2000
_proj = pl.pallas_call(
    _proj_body,
    grid=(B, M // _BM),
    in_specs=[
        pl.BlockSpec((1, _BM, FIN), lambda b, i: (b, i, 0)),
        pl.BlockSpec((FIN, K), lambda b, i: (0, 0)),
    ],
    out_specs=pl.BlockSpec((1, _BM, K), lambda b, i: (b, i, 0)),
    out_shape=jax.ShapeDtypeStruct((B, M, K), jnp.float32),
)


def kernel(x, edge_index, w_gcn_0, w_gcn_1, w_rlu):
    row = edge_index[0].astype(jnp.int32)
    col = edge_index[1].astype(jnp.int32)
    wm = w_gcn_0.reshape(FIN, K)
    q = _proj(x, wm)
    z = jnp.transpose(q, (2, 1, 0))
    z = jnp.pad(z, ((0, 0), (0, M_P - M), (0, 0)))
    za = jnp.concatenate([z[0], z[1]], axis=1)
    zb = jnp.concatenate([z[2], jnp.zeros_like(z[2])], axis=1)
    g = w_gcn_1[:, 0]
    r = w_rlu[0, 0, 0]
    cons = jnp.stack([
        jnp.full((16,), r, jnp.float32),
        jnp.full((16,), g[0] - g[2], jnp.float32),
        jnp.full((16,), g[1], jnp.float32),
        jnp.full((16,), 2.0 * g[2], jnp.float32),
    ])
    outp = _sc_cheb(row, col, za, zb, cons)
    return jnp.transpose(outp[:M, :8])


# speedup vs baseline: 36.1923x; 1.1838x over previous
import functools

import jax
import jax.numpy as jnp
from jax import lax
from jax.experimental import pallas as pl
from jax.experimental.pallas import tpu as pltpu
from jax.experimental.pallas import tpu_sc as plsc

M = 10000
E = 320000
FIN = 128
B = 8
K = 3

NT = 16
SR = 640
M_P = SR * NT
PTE = E // NT
CH = 800
NCH = PTE // CH

_MAGIC = 0x5F3759DF

_mesh = plsc.VectorSubcoreMesh(
    core_axis_name="c", subcore_axis_name="s", num_cores=1, num_subcores=NT
)

_sc_scratch = [
    pltpu.VMEM_SHARED((M_P,), jnp.float32),
    pltpu.VMEM_SHARED((M_P,), jnp.float32),
    pltpu.VMEM_SHARED((M_P, 16), jnp.float32),
    pltpu.VMEM_SHARED((M_P, 16), jnp.float32),
    pltpu.VMEM_SHARED((M_P, 16), jnp.float32),
    pltpu.VMEM_SHARED((M_P, 16), jnp.float32),
    pltpu.VMEM((CH,), jnp.int32),
    pltpu.VMEM((CH,), jnp.int32),
    pltpu.VMEM((CH, 16), jnp.float32),
    pltpu.VMEM((CH,), jnp.float32),
    pltpu.VMEM((SR,), jnp.float32),
    pltpu.VMEM((SR, 16), jnp.float32),
    pltpu.VMEM((SR, 16), jnp.float32),
    pltpu.VMEM((SR, 16), jnp.float32),
    pltpu.VMEM((SR, 16), jnp.float32),
    pltpu.VMEM((SR, 16), jnp.float32),
    pltpu.VMEM((SR, 16), jnp.float32),
    pltpu.VMEM((SR,), jnp.float32),
    pltpu.VMEM((SR,), jnp.float32),
    pltpu.VMEM((4, 16), jnp.float32),
]


@functools.partial(
    pl.kernel,
    out_type=jax.ShapeDtypeStruct((M_P, 16), jnp.float32),
    mesh=_mesh,
    scratch_types=_sc_scratch,
    compiler_params=pltpu.CompilerParams(
        needs_layout_passes=False, use_tc_tiling_on_sc=False
    ),
)
def _sc_cheb(rows_hbm, cols_hbm, za_hbm, zb_hbm, cons_hbm, out_hbm,
             sp_deg_r, sp_deg_c, sp_in, sp_acc1, sp_acc2, sp_h,
             ridx, cidx, gbuf, ones_v, zb1, zeros_v,
             wa, wb, wc, invr16, invc16, wdr, wdc, cons_v):
    t = lax.axis_index("s")
    ebase = t * PTE
    rbase = t * SR
    i16 = lax.iota(jnp.int32, 16)
    perm8 = jnp.bitwise_and(i16 + 8, 15)

    def rot8(v):
        return jnp.take_along_axis(
            v, perm8, axis=0, mode=lax.GatherScatterMode.PROMISE_IN_BOUNDS
        )

    def fill1(ref, n, val):
        def bd(i, _):
            ref[pl.ds(i * 16, 16)] = jnp.full((16,), val, jnp.float32)
            return 0
        lax.fori_loop(0, n, bd, 0)

    def fill2(ref, val):
        def bd(i, _):
            ref[i] = jnp.full((16,), val, jnp.float32)
            return 0
        lax.fori_loop(0, SR, bd, 0)

    def stripe(sp2d):
        return sp2d.at[pl.ds(rbase, SR)]

    def zero_acc(sp2d):
        pltpu.sync_copy(zeros_v, stripe(sp2d))

    def ew(body):
        def bd(i, _):
            body(i)
            return 0
        lax.fori_loop(0, SR, bd, 0)

    def s_pass(acc2d):
        def ch_body(ci, _):
            off = ebase + ci * CH
            pltpu.sync_copy(cols_hbm.at[pl.ds(off, CH)], cidx)
            pltpu.sync_copy(sp_in.at[cidx], gbuf)
            pltpu.sync_copy(rows_hbm.at[pl.ds(off, CH)], ridx)
            pltpu.sync_copy(gbuf, acc2d.at[ridx], add=True)
            return 0
        lax.fori_loop(0, NCH, ch_body, 0)

    fill1(ones_v, CH // 16, 1.0)
    fill1(zb1, SR // 16, 0.0)
    fill2(zeros_v, 0.0)
    pltpu.sync_copy(cons_hbm, cons_v)
    pltpu.sync_copy(zb1, sp_deg_r.at[pl.ds(rbase, SR)])
    pltpu.sync_copy(zb1, sp_deg_c.at[pl.ds(rbase, SR)])
    plsc.subcore_barrier()

    def deg_chunk(ci, _):
        off = ebase + ci * CH
        pltpu.sync_copy(rows_hbm.at[pl.ds(off, CH)], ridx)
        pltpu.sync_copy(cols_hbm.at[pl.ds(off, CH)], cidx)
        pltpu.sync_copy(ones_v, sp_deg_r.at[ridx], add=True)
        pltpu.sync_copy(ones_v, sp_deg_c.at[cidx], add=True)
        return 0
    lax.fori_loop(0, NCH, deg_chunk, 0)
    plsc.subcore_barrier()

    pltpu.sync_copy(sp_deg_r.at[pl.ds(rbase, SR)], wdr)
    pltpu.sync_copy(sp_deg_c.at[pl.ds(rbase, SR)], wdc)

    def inv_ref(ref):
        def bd(i, _):
            s = pl.ds(i * 16, 16)
            x = jnp.maximum(ref[s], 1.0)
            yi = (jnp.full((16,), _MAGIC, jnp.int32)
                  - lax.shift_right_logical(plsc.bitcast(x, jnp.int32), 1))
            y = plsc.bitcast(yi, jnp.float32)
            y = y * (1.5 - 0.5 * x * y * y)
            y = y * (1.5 - 0.5 * x * y * y)
            y = y * (1.5 - 0.5 * x * y * y)
            ref[s] = y
            return 0
        lax.fori_loop(0, SR // 16, bd, 0)
    inv_ref(wdr)
    inv_ref(wdc)

    def expand(i, _):
        sp = jnp.full((16,), i, jnp.int32)
        vr = plsc.load_gather(wdr, [sp])
        vc = plsc.load_gather(wdc, [sp])
        invr16[i] = vr
        invc16[i] = vc
        return 0
    lax.fori_loop(0, SR, expand, 0)

    rr = cons_v[0]
    c1 = cons_v[1]
    c2 = cons_v[2]
    c3 = cons_v[3]

    pltpu.sync_copy(za_hbm.at[pl.ds(rbase, SR)], wa)
    ew(lambda i: wa.__setitem__(i, wa[i] * invc16[i]))
    pltpu.sync_copy(wa, stripe(sp_in))
    zero_acc(sp_acc1)
    plsc.subcore_barrier()
    s_pass(sp_acc1)
    plsc.subcore_barrier()

    pltpu.sync_copy(stripe(sp_acc1), wa)
    ew(lambda i: wb.__setitem__(i, invr16[i] * invc16[i] * rot8(wa[i])))
    pltpu.sync_copy(wb, stripe(sp_in))
    zero_acc(sp_acc2)
    plsc.subcore_barrier()
    s_pass(sp_acc2)
    plsc.subcore_barrier()

    pltpu.sync_copy(zb_hbm.at[pl.ds(rbase, SR)], wa)
    pltpu.sync_copy(stripe(sp_acc1), wb)
    pltpu.sync_copy(stripe(sp_acc2), wc)
    ew(lambda i: wa.__setitem__(
        i, jnp.maximum(
            wa[i] - invr16[i] * wb[i] + 2.0 * invr16[i] * wc[i] + rr, 0.0)))
    pltpu.sync_copy(wa, stripe(sp_h))
    ew(lambda i: wb.__setitem__(i, invc16[i] * wa[i]))
    pltpu.sync_copy(wb, stripe(sp_in))
    zero_acc(sp_acc1)
    plsc.subcore_barrier()
    s_pass(sp_acc1)
    plsc.subcore_barrier()

    pltpu.sync_copy(stripe(sp_acc1), wb)
    ew(lambda i: wb.__setitem__(i, invr16[i] * invc16[i] * wb[i]))
    pltpu.sync_copy(wb, stripe(sp_in))
    zero_acc(sp_acc2)
    plsc.subcore_barrier()
    s_pass(sp_acc2)
    plsc.subcore_barrier()

    pltpu.sync_copy(stripe(sp_h), wa)
    pltpu.sync_copy(stripe(sp_acc1), wb)
    pltpu.sync_copy(stripe(sp_acc2), wc)
    ew(lambda i: wa.__setitem__(
        i, jnp.maximum(
            c1 * wa[i] + invr16[i] * (c3 * wc[i] - c2 * wb[i]) + rr, 0.0)))
    pltpu.sync_copy(wa, out_hbm.at[pl.ds(rbase, SR)])


def _proj_body(x_ref, w_ref, za_ref, zb_ref):
    bm = x_ref.shape[1]
    xall = x_ref[...].reshape(B * bm, FIN)
    v = lax.dot_general(
        xall, w_ref[...], (((1,), (0,)), ((), ())),
        preferred_element_type=jnp.float32,
    )
    v1 = [v[b * bm:(b + 1) * bm, 1:2] for b in range(B)]
    v2 = [v[b * bm:(b + 1) * bm, 2:3] for b in range(B)]
    v02 = [v[b * bm:(b + 1) * bm, 0:1] - v[b * bm:(b + 1) * bm, 2:3]
           for b in range(B)]
    za_ref[...] = jnp.concatenate(v1 + v2, axis=1)
    zb_ref[...] = jnp.concatenate(
        v02 + [jnp.zeros((bm, 8), jnp.float32)], axis=1)


_BM = 1000
_proj = pl.pallas_call(
    _proj_body,
    grid=(M // _BM,),
    in_specs=[
        pl.BlockSpec((B, _BM, FIN), lambda i: (0, i, 0)),
        pl.BlockSpec((FIN, K), lambda i: (0, 0)),
    ],
    out_specs=[
        pl.BlockSpec((_BM, 16), lambda i: (i, 0)),
        pl.BlockSpec((_BM, 16), lambda i: (i, 0)),
    ],
    out_shape=[
        jax.ShapeDtypeStruct((M, 16), jnp.float32),
        jax.ShapeDtypeStruct((M, 16), jnp.float32),
    ],
)


def kernel(x, edge_index, w_gcn_0, w_gcn_1, w_rlu):
    row = edge_index[0].astype(jnp.int32)
    col = edge_index[1].astype(jnp.int32)
    wm = w_gcn_0.reshape(FIN, K)
    za, zb = _proj(x, wm)
    za = jnp.pad(za, ((0, M_P - M), (0, 0)))
    zb = jnp.pad(zb, ((0, M_P - M), (0, 0)))
    g = w_gcn_1[:, 0]
    r = w_rlu[0, 0, 0]
    cons = jnp.stack([
        jnp.full((16,), r, jnp.float32),
        jnp.full((16,), g[0] - g[2], jnp.float32),
        jnp.full((16,), g[1], jnp.float32),
        jnp.full((16,), 2.0 * g[2], jnp.float32),
    ])
    outp = _sc_cheb(row, col, za, zb, cons)
    return jnp.transpose(outp[:M, :8])
